# baseline (speedup 1.0000x reference)
import functools

import jax
import jax.numpy as jnp
from jax import lax
from jax.experimental import pallas as pl
from jax.experimental.pallas import tpu as pltpu
from jax.experimental.pallas import tpu_sc as plsc

BATCH = 16384
EMBED_DIM = 64
CHUNK = 128


def _make_kernel():
    info = plsc.get_sparse_core_info()
    num_cores, num_subcores = info.num_cores, info.num_subcores
    nw = num_cores * num_subcores
    b_per_w = BATCH // nw
    n_chunks = b_per_w // CHUNK

    out_t = tuple(
        jax.ShapeDtypeStruct((BATCH, EMBED_DIM), jnp.float32) for _ in range(4)
    )

    @functools.partial(
        pl.kernel,
        mesh=plsc.VectorSubcoreMesh(core_axis_name="c", subcore_axis_name="s"),
        out_type=out_t,
        compiler_params=pltpu.CompilerParams(use_tc_tiling_on_sc=False),
        scratch_types=[
            pltpu.VMEM((n_chunks, CHUNK), jnp.int32),
            pltpu.VMEM((n_chunks, CHUNK), jnp.int32),
            pltpu.VMEM((n_chunks, CHUNK), jnp.int32),
            pltpu.VMEM((n_chunks, CHUNK), jnp.int32),
            pltpu.VMEM((b_per_w, EMBED_DIM), jnp.float32),
            pltpu.VMEM((b_per_w, EMBED_DIM), jnp.float32),
            pltpu.SemaphoreType.DMA,
            pltpu.SemaphoreType.DMA,
            pltpu.SemaphoreType.DMA,
            pltpu.SemaphoreType.DMA,
            pltpu.SemaphoreType.DMA,
        ],
    )
    def k(head_h, rel_h, tail_h, ts_h, ent_t, rel_t, ts_t,
          out0, out1, out2, out3,
          ih, ir, it, its, rows0, rows1,
          isem, gsem0, gsem1, wsem0, wsem1):
        wid = lax.axis_index("s") * num_cores + lax.axis_index("c")
        row_base = wid * b_per_w
        chunk_base = wid * n_chunks

        idx_loads = [
            pltpu.async_copy(src.at[pl.ds(chunk_base, n_chunks)], dst, isem)
            for src, dst in ((head_h, ih), (rel_h, ir), (tail_h, it), (ts_h, its))
        ]
        for cp in idx_loads:
            cp.wait()

        lookups = (
            (ent_t, ih, out0),
            (rel_t, ir, out1),
            (ent_t, it, out2),
            (ts_t, its, out3),
        )
        bufs = (rows0, rows1)
        gsems = (gsem0, gsem1)
        wsems = (wsem0, wsem1)
        pending_wb = [None, None]

        for i, (table, idx, out) in enumerate(lookups):
            slot = i % 2
            buf = bufs[slot]
            if pending_wb[slot] is not None:
                pending_wb[slot].wait()
                pending_wb[slot] = None
            gathers = [
                pltpu.async_copy(
                    table.at[idx.at[c]],
                    buf.at[pl.ds(c * CHUNK, CHUNK)],
                    gsems[slot],
                )
                for c in range(n_chunks)
            ]
            for cp in gathers:
                cp.wait()
            pending_wb[slot] = pltpu.async_copy(
                buf, out.at[pl.ds(row_base, b_per_w)], wsems[slot]
            )

        for slot in range(2):
            if pending_wb[slot] is not None:
                pending_wb[slot].wait()

    return k


_sc_lookup = _make_kernel()


def kernel(head, relation, tail, timestamp, entity_table, relation_table, timestamp_table):
    idx2 = lambda a: a.reshape(BATCH // CHUNK, CHUNK)
    return _sc_lookup(
        idx2(head), idx2(relation), idx2(tail), idx2(timestamp),
        entity_table, relation_table, timestamp_table,
    )


# speedup vs baseline: 1.0017x; 1.0017x over previous
import functools

import jax
import jax.numpy as jnp
from jax import lax
from jax.experimental import pallas as pl
from jax.experimental.pallas import tpu as pltpu
from jax.experimental.pallas import tpu_sc as plsc

BATCH = 16384
EMBED_DIM = 64
CHUNK = 128
NBUF = 8


def _make_kernel():
    info = plsc.get_sparse_core_info()
    num_cores, num_subcores = info.num_cores, info.num_subcores
    nw = num_cores * num_subcores
    b_per_w = BATCH // nw
    n_chunks = b_per_w // CHUNK
    n_tasks = 4 * n_chunks

    out_t = tuple(
        jax.ShapeDtypeStruct((BATCH, EMBED_DIM), jnp.float32) for _ in range(4)
    )

    scratch = (
        [pltpu.VMEM((n_chunks, CHUNK), jnp.int32) for _ in range(4)]
        + [pltpu.VMEM((CHUNK, EMBED_DIM), jnp.float32) for _ in range(NBUF)]
        + [pltpu.SemaphoreType.DMA]
        + [pltpu.SemaphoreType.DMA for _ in range(NBUF)]
        + [pltpu.SemaphoreType.DMA for _ in range(NBUF)]
    )

    @functools.partial(
        pl.kernel,
        mesh=plsc.VectorSubcoreMesh(core_axis_name="c", subcore_axis_name="s"),
        out_type=out_t,
        compiler_params=pltpu.CompilerParams(use_tc_tiling_on_sc=False),
        scratch_types=scratch,
    )
    def k(head_h, rel_h, tail_h, ts_h, ent_t, rel_t, ts_t,
          out0, out1, out2, out3, *sc):
        idx_bufs = sc[0:4]
        bufs = sc[4:4 + NBUF]
        isem = sc[4 + NBUF]
        gsems = sc[5 + NBUF:5 + 2 * NBUF]
        wsems = sc[5 + 2 * NBUF:5 + 3 * NBUF]

        wid = lax.axis_index("s") * num_cores + lax.axis_index("c")
        row_base = wid * b_per_w
        chunk_base = wid * n_chunks

        idx_loads = [
            pltpu.async_copy(src.at[pl.ds(chunk_base, n_chunks)], dst, isem)
            for src, dst in zip((head_h, rel_h, tail_h, ts_h), idx_bufs)
        ]
        for cp in idx_loads:
            cp.wait()

        tables = (ent_t, rel_t, ent_t, ts_t)
        outs = (out0, out1, out2, out3)
        tasks = [
            (idx_bufs[l].at[c], tables[l],
             outs[l].at[pl.ds(row_base + c * CHUNK, CHUNK)])
            for l in range(4) for c in range(n_chunks)
        ]

        gather_d = [None] * NBUF
        wb_d = [None] * NBUF

        def start(t):
            s = t % NBUF
            if wb_d[s] is not None:
                wb_d[s].wait()
            idx, table, _ = tasks[t]
            gather_d[s] = pltpu.async_copy(table.at[idx], bufs[s], gsems[s])

        def finish(t):
            s = t % NBUF
            gather_d[s].wait()
            _, _, dst = tasks[t]
            wb_d[s] = pltpu.async_copy(bufs[s], dst, wsems[s])

        depth = min(NBUF, n_tasks)
        for t in range(depth):
            start(t)
        for t in range(n_tasks):
            finish(t)
            nxt = t + depth
            if nxt < n_tasks:
                start(nxt)
        for s in range(NBUF):
            if wb_d[s] is not None:
                wb_d[s].wait()

    return k


_sc_lookup = _make_kernel()


def kernel(head, relation, tail, timestamp, entity_table, relation_table, timestamp_table):
    idx2 = lambda a: a.reshape(BATCH // CHUNK, CHUNK)
    return _sc_lookup(
        idx2(head), idx2(relation), idx2(tail), idx2(timestamp),
        entity_table, relation_table, timestamp_table,
    )


# speedup vs baseline: 1.5557x; 1.5531x over previous
import functools

import jax
import jax.numpy as jnp
from jax import lax
from jax.experimental import pallas as pl
from jax.experimental.pallas import tpu as pltpu
from jax.experimental.pallas import tpu_sc as plsc

BATCH = 16384
EMBED_DIM = 64
CHUNK = 128
LANES = 16


def _make_kernel():
    info = plsc.get_sparse_core_info()
    num_cores, num_subcores = info.num_cores, info.num_subcores
    nw = num_cores * num_subcores
    b_per_w = BATCH // nw
    n_chunks = b_per_w // CHUNK

    out_t = tuple(
        jax.ShapeDtypeStruct((BATCH, EMBED_DIM), jnp.float32) for _ in range(4)
    )

    scratch = (
        [pltpu.VMEM((b_per_w,), jnp.int32) for _ in range(4)]
        + [pltpu.VMEM((CHUNK, EMBED_DIM), jnp.float32)]
        + [pltpu.SemaphoreType.DMA,
           pltpu.SemaphoreType.DMA,
           pltpu.SemaphoreType.DMA]
    )

    @functools.partial(
        pl.kernel,
        mesh=plsc.VectorSubcoreMesh(core_axis_name="c", subcore_axis_name="s"),
        out_type=out_t,
        compiler_params=pltpu.CompilerParams(needs_layout_passes=False),
        scratch_types=scratch,
    )
    def k(head_h, rel_h, tail_h, ts_h, ent_t, rel_t, ts_t,
          out0, out1, out2, out3, *sc):
        idx_refs = sc[0:4]
        row_buf = sc[4]
        isem, gsem, wsem = sc[5:8]

        wid = lax.axis_index("s") * num_cores + lax.axis_index("c")
        row_base = wid * b_per_w

        idx_loads = [
            pltpu.async_copy(src.at[pl.ds(row_base, b_per_w)], dst, isem)
            for src, dst in zip((head_h, rel_h, tail_h, ts_h), idx_refs)
        ]
        for cp in idx_loads:
            cp.wait()

        tables = (ent_t, rel_t, ent_t, ts_t)
        outs = (out0, out1, out2, out3)

        for l in range(4):
            table, idx_ref, out = tables[l], idx_refs[l], outs[l]

            def chunk_body(c, _, table=table, idx_ref=idx_ref, out=out):
                off = c * CHUNK
                copies = []
                for g in range(CHUNK // LANES):
                    iv = idx_ref[pl.ds(off + g * LANES, LANES)]
                    for j in range(LANES):
                        copies.append(pltpu.async_copy(
                            table.at[iv[j]],
                            row_buf.at[g * LANES + j],
                            gsem,
                        ))
                for cp in copies:
                    cp.wait()
                pltpu.async_copy(
                    row_buf, out.at[pl.ds(row_base + off, CHUNK)], wsem
                ).wait()
                return 0

            lax.fori_loop(0, n_chunks, chunk_body, 0)

    return k


_sc_lookup = _make_kernel()


def kernel(head, relation, tail, timestamp, entity_table, relation_table, timestamp_table):
    return _sc_lookup(
        head, relation, tail, timestamp,
        entity_table, relation_table, timestamp_table,
    )


# speedup vs baseline: 1.5560x; 1.0002x over previous
import functools

import jax
import jax.numpy as jnp
from jax import lax
from jax.experimental import pallas as pl
from jax.experimental.pallas import tpu as pltpu
from jax.experimental.pallas import tpu_sc as plsc

BATCH = 16384
EMBED_DIM = 64
CHUNK = 128
LANES = 16


def _make_kernel():
    info = plsc.get_sparse_core_info()
    num_cores, num_subcores = info.num_cores, info.num_subcores
    nw = num_cores * num_subcores
    b_per_w = BATCH // nw
    n_chunks = b_per_w // CHUNK

    out_t = tuple(
        jax.ShapeDtypeStruct((BATCH, EMBED_DIM), jnp.float32) for _ in range(4)
    )

    scratch = (
        [pltpu.VMEM((b_per_w,), jnp.int32) for _ in range(4)]
        + [pltpu.VMEM((CHUNK, EMBED_DIM), jnp.float32)]
        + [pltpu.SemaphoreType.DMA,
           pltpu.SemaphoreType.DMA,
           pltpu.SemaphoreType.DMA]
    )

    @functools.partial(
        pl.kernel,
        mesh=plsc.VectorSubcoreMesh(core_axis_name="c", subcore_axis_name="s"),
        out_type=out_t,
        compiler_params=pltpu.CompilerParams(needs_layout_passes=False),
        scratch_types=scratch,
    )
    def k(head_h, rel_h, tail_h, ts_h, ent_t, rel_t, ts_t,
          out0, out1, out2, out3, *sc):
        idx_refs = sc[0:4]
        row_buf = sc[4]
        isem, gsem, wsem = sc[5:8]

        wid = lax.axis_index("s") * num_cores + lax.axis_index("c")
        row_base = wid * b_per_w

        idx_loads = [
            pltpu.async_copy(src.at[pl.ds(row_base, b_per_w)], dst, isem)
            for src, dst in zip((head_h, rel_h, tail_h, ts_h), idx_refs)
        ]
        for cp in idx_loads:
            cp.wait()

        tables = (ent_t, rel_t, ent_t, ts_t)
        outs = (out0, out1, out2, out3)

        for l in range(4):
            table, idx_ref, out = tables[l], idx_refs[l], outs[l]

            def chunk_body(c, _, table=table, idx_ref=idx_ref, out=out):
                off = c * CHUNK
                for g in range(CHUNK // LANES):
                    iv = idx_ref[pl.ds(off + g * LANES, LANES)]
                    for j in range(LANES):
                        pltpu.async_copy(
                            table.at[iv[j]],
                            row_buf.at[g * LANES + j],
                            gsem,
                        )
                pltpu.make_async_copy(
                    table.at[pl.ds(0, CHUNK)], row_buf, gsem
                ).wait()
                pltpu.async_copy(
                    row_buf, out.at[pl.ds(row_base + off, CHUNK)], wsem
                ).wait()
                return 0

            lax.fori_loop(0, n_chunks, chunk_body, 0)

    return k


_sc_lookup = _make_kernel()


def kernel(head, relation, tail, timestamp, entity_table, relation_table, timestamp_table):
    return _sc_lookup(
        head, relation, tail, timestamp,
        entity_table, relation_table, timestamp_table,
    )
